# Initial kernel scaffold; baseline (speedup 1.0000x reference)
import jax
import jax.numpy as jnp
from jax.experimental import pallas as pl


def kernel(class_ids, table):
    raise NotImplementedError("write your pallas kernel here")


# speedup vs baseline: 2.3728x; 2.3728x over previous
import functools

import jax
import jax.numpy as jnp
from jax import lax
from jax.experimental import pallas as pl
from jax.experimental.pallas import tpu as pltpu
from jax.experimental.pallas import tpu_sc as plsc

NUM_WORKERS = 32
CHUNK = 128


def kernel(class_ids, table):
    B, T = class_ids.shape
    V, D = table.shape
    total = B * T
    per_w = total // NUM_WORKERS
    rows_per_w = per_w // CHUNK
    ids2d = class_ids.reshape(total // CHUNK, CHUNK).astype(jnp.int32)

    mesh = plsc.VectorSubcoreMesh(core_axis_name="c", subcore_axis_name="s")

    @functools.partial(
        pl.kernel,
        out_type=jax.ShapeDtypeStruct((total, D), jnp.float32),
        mesh=mesh,
        scratch_types=[
            pltpu.VMEM((rows_per_w, CHUNK), jnp.int32),
            pltpu.VMEM((CHUNK, D), jnp.float32),
            pltpu.SemaphoreType.DMA,
        ],
    )
    def sc_gather(ids_hbm, table_hbm, out_hbm, idx_v, buf_v, sem):
        wid = lax.axis_index("s") * 2 + lax.axis_index("c")
        pltpu.sync_copy(ids_hbm.at[pl.ds(wid * rows_per_w, rows_per_w)], idx_v)
        base = wid * per_w

        @pl.loop(0, rows_per_w)
        def _(j):
            pltpu.async_copy(table_hbm.at[idx_v.at[j]], buf_v, sem).wait()
            pltpu.sync_copy(buf_v, out_hbm.at[pl.ds(base + j * CHUNK, CHUNK)])

    out = sc_gather(ids2d, table)
    return out.reshape(B, T, D)


# speedup vs baseline: 2.4221x; 1.0208x over previous
import functools

import jax
import jax.numpy as jnp
from jax import lax
from jax.experimental import pallas as pl
from jax.experimental.pallas import tpu as pltpu
from jax.experimental.pallas import tpu_sc as plsc

NUM_WORKERS = 32
CHUNK = 128
NBUF = 4
SKEW = 2


def kernel(class_ids, table):
    B, T = class_ids.shape
    V, D = table.shape
    total = B * T
    per_w = total // NUM_WORKERS
    n_chunks = per_w // CHUNK
    ids2d = class_ids.reshape(total // CHUNK, CHUNK).astype(jnp.int32)

    mesh = plsc.VectorSubcoreMesh(core_axis_name="c", subcore_axis_name="s")

    @functools.partial(
        pl.kernel,
        out_type=jax.ShapeDtypeStruct((total, D), jnp.float32),
        mesh=mesh,
        scratch_types=[
            pltpu.VMEM((n_chunks, CHUNK), jnp.int32),
            [pltpu.VMEM((CHUNK, D), jnp.float32) for _ in range(NBUF)],
            pltpu.SemaphoreType.DMA((NBUF,)),
            pltpu.SemaphoreType.DMA((NBUF,)),
        ],
    )
    def sc_gather(ids_hbm, table_hbm, out_hbm, idx_v, bufs, gsem, ssem):
        wid = lax.axis_index("s") * 2 + lax.axis_index("c")
        pltpu.sync_copy(ids_hbm.at[pl.ds(wid * n_chunks, n_chunks)], idx_v)
        base = wid * per_w

        def gather(j, b):
            return pltpu.make_async_copy(
                table_hbm.at[idx_v.at[j]], bufs[b], gsem.at[b])

        def scatter(j, b):
            return pltpu.make_async_copy(
                bufs[b], out_hbm.at[pl.ds(base + j * CHUNK, CHUNK)], ssem.at[b])

        for b in range(NBUF):
            gather(b, b).start()

        @pl.loop(0, n_chunks // NBUF)
        def _(g):
            for b in range(NBUF):
                j = g * NBUF + b
                gather(j, b).wait()
                scatter(j, b).start()
                jp = j - SKEW
                bp = (b - SKEW) % NBUF

                @pl.when(jp >= 0)
                def _():
                    scatter(jp, bp).wait()

                    @pl.when(jp + NBUF < n_chunks)
                    def _():
                        gather(jp + NBUF, bp).start()

        for j in range(n_chunks - SKEW, n_chunks):
            scatter(j, j % NBUF).wait()

    out = sc_gather(ids2d, table)
    return out.reshape(B, T, D)


# speedup vs baseline: 9.9985x; 4.1281x over previous
import functools

import jax
import jax.numpy as jnp
from jax import lax
from jax.experimental import pallas as pl
from jax.experimental.pallas import tpu as pltpu
from jax.experimental.pallas import tpu_sc as plsc

NUM_WORKERS = 32
CHUNK = 128
NBUF = 4
SKEW = 2


def kernel(class_ids, table):
    B, T = class_ids.shape
    V, D = table.shape
    total = B * T
    per_w = total // NUM_WORKERS
    n_chunks = per_w // CHUNK
    ids2d = class_ids.reshape(total // CHUNK, CHUNK).astype(jnp.int32)

    mesh = plsc.VectorSubcoreMesh(core_axis_name="c", subcore_axis_name="s")

    @functools.partial(
        pl.kernel,
        out_type=jax.ShapeDtypeStruct((total, D), jnp.float32),
        mesh=mesh,
        scratch_types=[
            pltpu.VMEM((n_chunks, CHUNK), jnp.int32),
            [pltpu.VMEM((CHUNK, D), jnp.float32) for _ in range(NBUF)],
            pltpu.VMEM_SHARED((V, D), jnp.float32),
            pltpu.SemaphoreType.DMA((NBUF,)),
            pltpu.SemaphoreType.DMA((NBUF,)),
        ],
    )
    def sc_gather(ids_hbm, table_hbm, out_hbm, idx_v, bufs, table_sh, gsem, ssem):
        wid = lax.axis_index("s") * 2 + lax.axis_index("c")

        @pl.when(lax.axis_index("s") == 0)
        def _():
            pltpu.sync_copy(table_hbm, table_sh)

        plsc.subcore_barrier()

        pltpu.sync_copy(ids_hbm.at[pl.ds(wid * n_chunks, n_chunks)], idx_v)
        base = wid * per_w

        def gather(j, b):
            return pltpu.make_async_copy(
                table_sh.at[idx_v.at[j]], bufs[b], gsem.at[b])

        def scatter(j, b):
            return pltpu.make_async_copy(
                bufs[b], out_hbm.at[pl.ds(base + j * CHUNK, CHUNK)], ssem.at[b])

        for b in range(NBUF):
            gather(b, b).start()

        @pl.loop(0, n_chunks // NBUF)
        def _(g):
            for b in range(NBUF):
                j = g * NBUF + b
                gather(j, b).wait()
                scatter(j, b).start()
                jp = j - SKEW
                bp = (b - SKEW) % NBUF

                @pl.when(jp >= 0)
                def _():
                    scatter(jp, bp).wait()

                    @pl.when(jp + NBUF < n_chunks)
                    def _():
                        gather(jp + NBUF, bp).start()

        for j in range(n_chunks - SKEW, n_chunks):
            scatter(j, j % NBUF).wait()

    out = sc_gather(ids2d, table)
    return out.reshape(B, T, D)
